# baseline (speedup 1.0000x reference)
import functools

import jax
import jax.numpy as jnp
from jax import lax
from jax.experimental import pallas as pl
from jax.experimental.pallas import tpu as pltpu
from jax.experimental.pallas import tpu_sc as plsc

N = 10000
E = 320000
D = 128
AVG_D_LOG = 3.5
EPS = 1e-5
ROWS = 1000

NC = 2
NS = 16
L = 16
NPT = 320
HNPT = 160
PSC = NS * NPT
NT = NC * NS
BE = 4000
NB = E // BE
G = 64
HB = BE + G + 16
DEGW = 176

_NEG = -3.0e38
_POS = 3.0e38


def _sc_agg_body(h_hbm, edge_hbm, sum_out, sq_out, max_out, min_out, deg_out,
                 srcblk0, dstblk0, srcblk1, dstblk1, hit_s, hit_d,
                 mbuf0, mbuf1, sumtab, sqtab, maxtab, mintab, degtab,
                 sem_e0, sem_e1, sem_g0, sem_g1):
    c = lax.axis_index("c")
    s = lax.axis_index("s")

    zero16 = jnp.zeros((L,), jnp.float32)
    neg16 = jnp.full((L,), _NEG, jnp.float32)
    pos16 = jnp.full((L,), _POS, jnp.float32)
    lane0 = lax.iota(jnp.int32, L) == 0
    one16 = jnp.ones((L,), jnp.float32)
    dump16 = jnp.full((L,), 0, jnp.int32)
    one16i = jnp.full((L,), 1, jnp.int32)
    zero16i = jnp.full((L,), 0, jnp.int32)

    ebufs = ((srcblk0, dstblk0, sem_e0), (srcblk1, dstblk1, sem_e1))
    gbufs = ((mbuf0, sem_g0), (mbuf1, sem_g1))

    for r in range(2):
        lo = c * PSC + s * NPT + r * HNPT

        def _init_tabs(i, _):
            rr = i // 8
            o = (i % 8) * L
            sumtab[rr, pl.ds(o, L)] = zero16
            sqtab[rr, pl.ds(o, L)] = zero16
            maxtab[rr, pl.ds(o, L)] = neg16
            mintab[rr, pl.ds(o, L)] = pos16
            return 0
        lax.fori_loop(0, HNPT * 8, _init_tabs, 0)

        def _init_deg(i, _):
            degtab[pl.ds(i * L, L)] = zero16
            return 0
        lax.fori_loop(0, DEGW // L, _init_deg, 0)

        pltpu.async_copy(edge_hbm.at[pl.ds(0, BE)], srcblk0, sem_e0)
        pltpu.async_copy(edge_hbm.at[pl.ds(E, BE)], dstblk0, sem_e0)

        def _do_block(eb, sb, db, se, nsb, ndb, nse):
            pltpu.make_async_copy(edge_hbm.at[pl.ds(0, BE)], sb, se).wait()
            pltpu.make_async_copy(edge_hbm.at[pl.ds(0, BE)], db, se).wait()

            @pl.when(eb + 1 < NB)
            def _pref():
                off2 = (eb + 1) * BE
                pltpu.async_copy(edge_hbm.at[pl.ds(off2, BE)], nsb, nse)
                pltpu.async_copy(edge_hbm.at[pl.ds(E + off2, BE)], ndb, nse)

            def _filt(i, hc):
                d = db[pl.ds(i * L, L)]
                sv = sb[pl.ds(i * L, L)]
                tl = d - lo
                msk = (tl >= 0) & (tl < HNPT)
                mi = jnp.where(msk, one16i, zero16i)
                pos = plsc.cumsum(mi) + jnp.full((L,), hc - 1, jnp.int32)
                plsc.store_scatter(hit_s, [pos], sv, mask=msk)
                plsc.store_scatter(hit_d, [pos], tl, mask=msk)
                return pos[15] + 1
            hc = lax.fori_loop(0, BE // L, _filt, 0)

            for tpad in range(4):
                hit_s[pl.ds(hc + tpad * L, L)] = dump16

            nch = (hc + G - 1) >> 6

            @pl.when(nch > 0)
            def _g0():
                pltpu.async_copy(h_hbm.at[hit_s.at[pl.ds(0, G)]], mbuf0,
                                 sem_g0)

            def _process(cur, mb, sg, nmb, nsg):
                cbase = cur * G
                pltpu.make_async_copy(h_hbm.at[hit_s.at[pl.ds(0, G)]], mb,
                                      sg).wait()

                @pl.when(cur + 1 < nch)
                def _gn():
                    pltpu.async_copy(
                        h_hbm.at[hit_s.at[pl.ds(cbase + G, G)]], nmb, nsg)

                tl0v = hit_d[pl.ds(cbase, L)]
                tl0 = tl0v[0]

                def _edge(e, carry):
                    tl, tlb = carry
                    nxt = hit_d[pl.ds(cbase + e + 1, L)]
                    tl_n = nxt[0]
                    tlb_n = jnp.full((L,), tl_n, jnp.int32)
                    plsc.addupdate_scatter(degtab, [tlb], one16, mask=lane0)
                    for j in range(D // L):
                        o = j * L
                        mv = mb[e, pl.ds(o, L)]
                        plsc.addupdate(sumtab.at[tl, pl.ds(o, L)], mv)
                        plsc.addupdate(sqtab.at[tl, pl.ds(o, L)], mv * mv)
                        a = maxtab[tl, pl.ds(o, L)]
                        maxtab[tl, pl.ds(o, L)] = jnp.maximum(a, mv)
                        b = mintab[tl, pl.ds(o, L)]
                        mintab[tl, pl.ds(o, L)] = jnp.minimum(b, mv)
                    return (tl_n, tlb_n)
                ne = jnp.minimum(G, hc - cbase)
                lax.fori_loop(0, ne, _edge,
                              (tl0, jnp.full((L,), tl0, jnp.int32)))

            def _cpair(cp, _):
                for b in range(2):
                    cur = cp * 2 + b
                    mb, sg = gbufs[b]
                    nmb, nsg = gbufs[1 - b]

                    @pl.when(cur < nch)
                    def _doc():
                        _process(cur, mb, sg, nmb, nsg)
                return 0
            lax.fori_loop(0, (nch + 1) >> 1, _cpair, 0)

        def _bpair(bp, _):
            for b in range(2):
                eb = bp * 2 + b
                sb, db, se = ebufs[b]
                nsb, ndb, nse = ebufs[1 - b]
                _do_block(eb, sb, db, se, nsb, ndb, nse)
            return 0
        lax.fori_loop(0, NB // 2, _bpair, 0)

        pltpu.sync_copy(sumtab, sum_out.at[pl.ds(lo, HNPT)])
        pltpu.sync_copy(sqtab, sq_out.at[pl.ds(lo, HNPT)])
        pltpu.sync_copy(maxtab, max_out.at[pl.ds(lo, HNPT)])
        pltpu.sync_copy(mintab, min_out.at[pl.ds(lo, HNPT)])
        pltpu.sync_copy(degtab, deg_out.at[pl.ds(((c * NS + s) * 2 + r) * DEGW, DEGW)])


def sc_aggregate(h, edge_index):
    mesh = plsc.VectorSubcoreMesh(core_axis_name="c", subcore_axis_name="s")
    k = functools.partial(
        pl.kernel,
        mesh=mesh,
        out_type=[
            jax.ShapeDtypeStruct((NT * NPT, D), jnp.float32),
            jax.ShapeDtypeStruct((NT * NPT, D), jnp.float32),
            jax.ShapeDtypeStruct((NT * NPT, D), jnp.float32),
            jax.ShapeDtypeStruct((NT * NPT, D), jnp.float32),
            jax.ShapeDtypeStruct((NT * 2 * DEGW,), jnp.float32),
        ],
        scratch_types=[
            pltpu.VMEM((BE,), jnp.int32),
            pltpu.VMEM((BE,), jnp.int32),
            pltpu.VMEM((BE,), jnp.int32),
            pltpu.VMEM((BE,), jnp.int32),
            pltpu.VMEM((HB,), jnp.int32),
            pltpu.VMEM((HB,), jnp.int32),
            pltpu.VMEM((G, D), jnp.float32),
            pltpu.VMEM((G, D), jnp.float32),
            pltpu.VMEM((HNPT, D), jnp.float32),
            pltpu.VMEM((HNPT, D), jnp.float32),
            pltpu.VMEM((HNPT, D), jnp.float32),
            pltpu.VMEM((HNPT, D), jnp.float32),
            pltpu.VMEM((DEGW,), jnp.float32),
            pltpu.SemaphoreType.DMA,
            pltpu.SemaphoreType.DMA,
            pltpu.SemaphoreType.DMA,
            pltpu.SemaphoreType.DMA,
        ],
        compiler_params=pltpu.CompilerParams(needs_layout_passes=False),
    )(_sc_agg_body)
    ssum, ssq, smax, smin, sdeg = k(h, edge_index.reshape(2 * E))
    s = ssum[:N]
    sq = ssq[:N]
    mx = smax[:N]
    mn = smin[:N]
    deg = sdeg.reshape(NT * 2, DEGW)[:, :HNPT].reshape(NT * NPT)[:N]
    return s, sq, mx, mn, deg


def _t1_body(sum_ref, sq_ref, mx_ref, mn_ref, deg_ref, wt_ref, b_ref,
             raw_ref, cs_ref, csq_ref, acc1, acc2):
    i = pl.program_id(0)
    deg = deg_ref[...]
    degc = jnp.maximum(deg, 1.0)
    mean = sum_ref[...] / degc
    meansq = sq_ref[...] / degc
    std = jnp.sqrt(jnp.maximum(meansq - mean * mean, 0.0) + EPS)
    has = deg > 0.0
    mx = jnp.where(has, mx_ref[...], 0.0)
    mn = jnp.where(has, mn_ref[...], 0.0)
    agg = jnp.concatenate([mean, mx, mn, std], axis=1)
    logd = jnp.log(degc + 1.0)
    hs = jnp.concatenate([agg, agg * (logd / AVG_D_LOG), agg * (AVG_D_LOG / logd)],
                         axis=1)
    raw = jnp.dot(hs, wt_ref[...], preferred_element_type=jnp.float32) + b_ref[...]
    raw_ref[...] = raw

    @pl.when(i == 0)
    def _init():
        acc1[...] = jnp.zeros_like(acc1)
        acc2[...] = jnp.zeros_like(acc2)

    acc1[...] += jnp.sum(raw, axis=0, keepdims=True)
    acc2[...] += jnp.sum(raw * raw, axis=0, keepdims=True)

    @pl.when(i == pl.num_programs(0) - 1)
    def _fin():
        cs_ref[...] = acc1[...]
        csq_ref[...] = acc2[...]


def _t2_body(raw_ref, h_ref, cs_ref, csq_ref, g_ref, bt_ref, out_ref):
    mu = cs_ref[...] / N
    var = csq_ref[...] / N - mu * mu
    inv = jax.lax.rsqrt(var + 1e-5)
    y = (raw_ref[...] - mu) * inv * g_ref[...] + bt_ref[...]
    out_ref[...] = jnp.maximum(y, 0.0) + h_ref[...]


def _posttrans(s, sq, mx, mn, deg, h, Wt, b, gamma, beta):
    grid = N // ROWS
    row = lambda i: (i, 0)
    fixed = lambda i: (0, 0)
    raw, cs, csq = pl.pallas_call(
        _t1_body,
        grid=(grid,),
        in_specs=[
            pl.BlockSpec((ROWS, D), row),
            pl.BlockSpec((ROWS, D), row),
            pl.BlockSpec((ROWS, D), row),
            pl.BlockSpec((ROWS, D), row),
            pl.BlockSpec((ROWS, 1), row),
            pl.BlockSpec((12 * D, D), fixed),
            pl.BlockSpec((1, D), fixed),
        ],
        out_specs=[
            pl.BlockSpec((ROWS, D), row),
            pl.BlockSpec((1, D), fixed),
            pl.BlockSpec((1, D), fixed),
        ],
        out_shape=[
            jax.ShapeDtypeStruct((N, D), jnp.float32),
            jax.ShapeDtypeStruct((1, D), jnp.float32),
            jax.ShapeDtypeStruct((1, D), jnp.float32),
        ],
        scratch_shapes=[
            pltpu.VMEM((1, D), jnp.float32),
            pltpu.VMEM((1, D), jnp.float32),
        ],
    )(s, sq, mx, mn, deg, Wt, b)
    out = pl.pallas_call(
        _t2_body,
        grid=(grid,),
        in_specs=[
            pl.BlockSpec((ROWS, D), row),
            pl.BlockSpec((ROWS, D), row),
            pl.BlockSpec((1, D), fixed),
            pl.BlockSpec((1, D), fixed),
            pl.BlockSpec((1, D), fixed),
            pl.BlockSpec((1, D), fixed),
        ],
        out_specs=pl.BlockSpec((ROWS, D), row),
        out_shape=jax.ShapeDtypeStruct((N, D), jnp.float32),
    )(raw, h, cs, csq, gamma, beta)
    return out


def kernel(h, edge_index, W, b, gamma, beta):
    s, sq, mx, mn, deg = sc_aggregate(h, edge_index)
    Wt = W.T.reshape(12 * D, D)
    return _posttrans(s, sq, mx, mn, deg.reshape(N, 1), h,
                      Wt, b.reshape(1, D), gamma.reshape(1, D),
                      beta.reshape(1, D))


# speedup vs baseline: 1.7894x; 1.7894x over previous
import functools

import jax
import jax.numpy as jnp
from jax import lax
from jax.experimental import pallas as pl
from jax.experimental.pallas import tpu as pltpu
from jax.experimental.pallas import tpu_sc as plsc

N = 10000
E = 320000
D = 128
AVG_D_LOG = 3.5
EPS = 1e-5
ROWS = 1000

NC = 2
NS = 16
L = 16
NPT = 320
HNPT = 160
PSC = NS * NPT
NT = NC * NS
BE = 6400
NB = E // BE
G = 64
HB = BE + G + 16
DEGW = 176

_NEG = -3.0e38
_POS = 3.0e38


def _sc_agg_body(h_hbm, edge_hbm, sum_out, sq_out, max_out, min_out, deg_out,
                 srcblk, dstblk, hit_s, hit_d,
                 mbuf, mbuf1, sumtab, sqtab, maxtab, mintab, degtab, sem,
                 sem1):
    gbufs = ((mbuf, sem), (mbuf1, sem1))
    c = lax.axis_index("c")
    s = lax.axis_index("s")

    zero16 = jnp.zeros((L,), jnp.float32)
    neg16 = jnp.full((L,), _NEG, jnp.float32)
    pos16 = jnp.full((L,), _POS, jnp.float32)
    lane0 = lax.iota(jnp.int32, L) == 0
    one16 = jnp.ones((L,), jnp.float32)
    dump16 = jnp.full((L,), 0, jnp.int32)
    one16i = jnp.full((L,), 1, jnp.int32)
    zero16i = jnp.full((L,), 0, jnp.int32)

    for r in range(2):
        lo = c * PSC + s * NPT + r * HNPT

        def _init_tabs(i, _):
            rr = i // 8
            o = (i % 8) * L
            sumtab[rr, pl.ds(o, L)] = zero16
            sqtab[rr, pl.ds(o, L)] = zero16
            maxtab[rr, pl.ds(o, L)] = neg16
            mintab[rr, pl.ds(o, L)] = pos16
            return 0
        lax.fori_loop(0, HNPT * 8, _init_tabs, 0)

        def _init_deg(i, _):
            degtab[pl.ds(i * L, L)] = zero16
            return 0
        lax.fori_loop(0, DEGW // L, _init_deg, 0)

        def _block(eb, _):
            off = eb * BE
            pltpu.sync_copy(edge_hbm.at[0, pl.ds(off, BE)], srcblk)
            pltpu.sync_copy(edge_hbm.at[1, pl.ds(off, BE)], dstblk)

            def _filt(i, hc):
                d = dstblk[pl.ds(i * L, L)]
                sv = srcblk[pl.ds(i * L, L)]
                tl = d - lo
                msk = (tl >= 0) & (tl < HNPT)
                mi = jnp.where(msk, one16i, zero16i)
                pos = plsc.cumsum(mi) + jnp.full((L,), hc - 1, jnp.int32)
                plsc.store_scatter(hit_s, [pos], sv, mask=msk)
                plsc.store_scatter(hit_d, [pos], tl, mask=msk)
                return pos[15] + 1
            hc = lax.fori_loop(0, BE // L, _filt, 0)

            for tpad in range(4):
                hit_s[pl.ds(hc + tpad * L, L)] = dump16

            nch = (hc + G - 1) >> 6

            @pl.when(nch > 0)
            def _g0():
                pltpu.async_copy(h_hbm.at[hit_s.at[pl.ds(0, G)]], mbuf, sem)

            def _chunk(ch, mb, sg, nmb, nsg):
                cbase = ch * G
                pltpu.make_async_copy(h_hbm.at[hit_s.at[pl.ds(0, G)]], mb,
                                      sg).wait()

                @pl.when(ch + 1 < nch)
                def _gn():
                    pltpu.async_copy(
                        h_hbm.at[hit_s.at[pl.ds(cbase + G, G)]], nmb, nsg)

                def _edge(e, _):
                    tlv = hit_d[pl.ds(cbase + e, L)]
                    tl = tlv[0]
                    plsc.addupdate_scatter(
                        degtab, [jnp.full((L,), tl, jnp.int32)],
                        one16, mask=lane0)
                    for j in range(D // L):
                        o = j * L
                        mv = mb[e, pl.ds(o, L)]
                        plsc.addupdate(sumtab.at[tl, pl.ds(o, L)], mv)
                        plsc.addupdate(sqtab.at[tl, pl.ds(o, L)], mv * mv)
                        a = maxtab[tl, pl.ds(o, L)]
                        maxtab[tl, pl.ds(o, L)] = jnp.maximum(a, mv)
                        b = mintab[tl, pl.ds(o, L)]
                        mintab[tl, pl.ds(o, L)] = jnp.minimum(b, mv)
                    return 0
                ne = jnp.minimum(G, hc - cbase)
                lax.fori_loop(0, ne, _edge, 0)

            def _cpair(cp, _):
                for b in range(2):
                    cur = cp * 2 + b
                    mb, sg = gbufs[b]
                    nmb, nsg = gbufs[1 - b]

                    @pl.when(cur < nch)
                    def _doc():
                        _chunk(cur, mb, sg, nmb, nsg)
                return 0
            lax.fori_loop(0, (nch + 1) >> 1, _cpair, 0)
            return 0
        lax.fori_loop(0, NB, _block, 0)

        pltpu.sync_copy(sumtab, sum_out.at[pl.ds(lo, HNPT)])
        pltpu.sync_copy(sqtab, sq_out.at[pl.ds(lo, HNPT)])
        pltpu.sync_copy(maxtab, max_out.at[pl.ds(lo, HNPT)])
        pltpu.sync_copy(mintab, min_out.at[pl.ds(lo, HNPT)])
        pltpu.sync_copy(degtab, deg_out.at[c, s, r])


def sc_aggregate(h, edge_index):
    mesh = plsc.VectorSubcoreMesh(core_axis_name="c", subcore_axis_name="s")
    k = functools.partial(
        pl.kernel,
        mesh=mesh,
        out_type=[
            jax.ShapeDtypeStruct((NT * NPT, D), jnp.float32),
            jax.ShapeDtypeStruct((NT * NPT, D), jnp.float32),
            jax.ShapeDtypeStruct((NT * NPT, D), jnp.float32),
            jax.ShapeDtypeStruct((NT * NPT, D), jnp.float32),
            jax.ShapeDtypeStruct((NC, NS, 2, DEGW), jnp.float32),
        ],
        scratch_types=[
            pltpu.VMEM((BE,), jnp.int32),
            pltpu.VMEM((BE,), jnp.int32),
            pltpu.VMEM((HB,), jnp.int32),
            pltpu.VMEM((HB,), jnp.int32),
            pltpu.VMEM((G, D), jnp.float32),
            pltpu.VMEM((G, D), jnp.float32),
            pltpu.VMEM((HNPT, D), jnp.float32),
            pltpu.VMEM((HNPT, D), jnp.float32),
            pltpu.VMEM((HNPT, D), jnp.float32),
            pltpu.VMEM((HNPT, D), jnp.float32),
            pltpu.VMEM((DEGW,), jnp.float32),
            pltpu.SemaphoreType.DMA,
            pltpu.SemaphoreType.DMA,
        ],
        compiler_params=pltpu.CompilerParams(needs_layout_passes=False),
    )(_sc_agg_body)
    ssum, ssq, smax, smin, sdeg = k(h, edge_index)
    s = ssum[:N]
    sq = ssq[:N]
    mx = smax[:N]
    mn = smin[:N]
    deg = sdeg[:, :, :, :HNPT].reshape(NT * NPT)[:N]
    return s, sq, mx, mn, deg


def _t1_body(sum_ref, sq_ref, mx_ref, mn_ref, deg_ref, wt_ref, b_ref,
             raw_ref, cs_ref, csq_ref, acc1, acc2):
    i = pl.program_id(0)
    deg = deg_ref[...]
    degc = jnp.maximum(deg, 1.0)
    mean = sum_ref[...] / degc
    meansq = sq_ref[...] / degc
    std = jnp.sqrt(jnp.maximum(meansq - mean * mean, 0.0) + EPS)
    has = deg > 0.0
    mx = jnp.where(has, mx_ref[...], 0.0)
    mn = jnp.where(has, mn_ref[...], 0.0)
    agg = jnp.concatenate([mean, mx, mn, std], axis=1)
    logd = jnp.log(degc + 1.0)
    hs = jnp.concatenate([agg, agg * (logd / AVG_D_LOG), agg * (AVG_D_LOG / logd)],
                         axis=1)
    raw = jnp.dot(hs, wt_ref[...], preferred_element_type=jnp.float32) + b_ref[...]
    raw_ref[...] = raw

    @pl.when(i == 0)
    def _init():
        acc1[...] = jnp.zeros_like(acc1)
        acc2[...] = jnp.zeros_like(acc2)

    acc1[...] += jnp.sum(raw, axis=0, keepdims=True)
    acc2[...] += jnp.sum(raw * raw, axis=0, keepdims=True)

    @pl.when(i == pl.num_programs(0) - 1)
    def _fin():
        cs_ref[...] = acc1[...]
        csq_ref[...] = acc2[...]


def _t2_body(raw_ref, h_ref, cs_ref, csq_ref, g_ref, bt_ref, out_ref):
    mu = cs_ref[...] / N
    var = csq_ref[...] / N - mu * mu
    inv = jax.lax.rsqrt(var + 1e-5)
    y = (raw_ref[...] - mu) * inv * g_ref[...] + bt_ref[...]
    out_ref[...] = jnp.maximum(y, 0.0) + h_ref[...]


def _posttrans(s, sq, mx, mn, deg, h, Wt, b, gamma, beta):
    grid = N // ROWS
    row = lambda i: (i, 0)
    fixed = lambda i: (0, 0)
    raw, cs, csq = pl.pallas_call(
        _t1_body,
        grid=(grid,),
        in_specs=[
            pl.BlockSpec((ROWS, D), row),
            pl.BlockSpec((ROWS, D), row),
            pl.BlockSpec((ROWS, D), row),
            pl.BlockSpec((ROWS, D), row),
            pl.BlockSpec((ROWS, 1), row),
            pl.BlockSpec((12 * D, D), fixed),
            pl.BlockSpec((1, D), fixed),
        ],
        out_specs=[
            pl.BlockSpec((ROWS, D), row),
            pl.BlockSpec((1, D), fixed),
            pl.BlockSpec((1, D), fixed),
        ],
        out_shape=[
            jax.ShapeDtypeStruct((N, D), jnp.float32),
            jax.ShapeDtypeStruct((1, D), jnp.float32),
            jax.ShapeDtypeStruct((1, D), jnp.float32),
        ],
        scratch_shapes=[
            pltpu.VMEM((1, D), jnp.float32),
            pltpu.VMEM((1, D), jnp.float32),
        ],
    )(s, sq, mx, mn, deg, Wt, b)
    out = pl.pallas_call(
        _t2_body,
        grid=(grid,),
        in_specs=[
            pl.BlockSpec((ROWS, D), row),
            pl.BlockSpec((ROWS, D), row),
            pl.BlockSpec((1, D), fixed),
            pl.BlockSpec((1, D), fixed),
            pl.BlockSpec((1, D), fixed),
            pl.BlockSpec((1, D), fixed),
        ],
        out_specs=pl.BlockSpec((ROWS, D), row),
        out_shape=jax.ShapeDtypeStruct((N, D), jnp.float32),
    )(raw, h, cs, csq, gamma, beta)
    return out


def kernel(h, edge_index, W, b, gamma, beta):
    s, sq, mx, mn, deg = sc_aggregate(h, edge_index)
    Wt = W.T.reshape(12 * D, D)
    return _posttrans(s, sq, mx, mn, deg.reshape(N, 1), h,
                      Wt, b.reshape(1, D), gamma.reshape(1, D),
                      beta.reshape(1, D))
